# baseline (speedup 1.0000x reference)
import jax
import jax.numpy as jnp
from jax import lax
from jax.experimental import pallas as pl
from jax.experimental.pallas import tpu as pltpu
from jax.experimental.pallas import tpu_sc as plsc
from jax.experimental.layout import (Format as _Format, Layout as _Layout,
                                     with_layout_constraint as _with_layout_constraint)

_VOCAB = 100000
_HID = 64
_MAXLEN = 200
_BATCH = 4096
_EPS = 1e-8
_NC = 2
_NS = 16
_NW = _NC * _NS
_NTOK = _BATCH * _MAXLEN
_TOK_PER_W = _NTOK // _NW
_CHUNK = 256
_NCHUNK = _TOK_PER_W // _CHUNK


def _gather_body(tok_hbm, word_hbm, rows_hbm,
                 idx0, idx1, buf0, buf1, sem0, sem1, semo):
    wid = lax.axis_index("s") * _NC + lax.axis_index("c")
    base = wid * _TOK_PER_W

    def issue(off, idx_v, buf_v, sem):
        pltpu.sync_copy(tok_hbm.at[pl.ds(off, _CHUNK)], idx_v)
        for j in range(_CHUNK // 128):
            pltpu.async_copy(word_hbm.at[idx_v.at[pl.ds(128 * j, 128)]],
                             buf_v.at[pl.ds(128 * j, 128)], sem)

    lane = lax.iota(jnp.int32, 16)
    col64 = jnp.full((16,), _HID, jnp.int32)

    def drain(off, idx_v, buf_v, sem):
        for j in range(_CHUNK // 128):
            pltpu.make_async_copy(word_hbm.at[idx_v.at[pl.ds(128 * j, 128)]],
                                  buf_v.at[pl.ds(128 * j, 128)], sem).wait()
        for k in range(_CHUNK // 16):
            tk = idx_v[pl.ds(16 * k, 16)]
            m = jnp.where(tk != 0, jnp.float32(1.0), jnp.float32(0.0))
            plsc.store_scatter(buf_v, [16 * k + lane, col64], m)
        pltpu.async_copy(buf_v, rows_hbm.at[pl.ds(off, _CHUNK)], semo).wait()

    issue(base, idx0, buf0, sem0)

    def chunk_body(i, c):
        slot = lax.rem(i, 2)

        @pl.when(jnp.logical_and(i + 1 < _NCHUNK, slot == 0))
        def _():
            issue(base + (i + 1) * _CHUNK, idx1, buf1, sem1)

        @pl.when(jnp.logical_and(i + 1 < _NCHUNK, slot == 1))
        def _():
            issue(base + (i + 1) * _CHUNK, idx0, buf0, sem0)

        off_i = base + i * _CHUNK

        @pl.when(slot == 0)
        def _():
            drain(off_i, idx0, buf0, sem0)

        @pl.when(slot == 1)
        def _():
            drain(off_i, idx1, buf1, sem1)

        return c

    lax.fori_loop(0, _NCHUNK, chunk_body, 0)


_gather = pl.kernel(
    _gather_body,
    mesh=plsc.VectorSubcoreMesh(core_axis_name="c", subcore_axis_name="s"),
    out_type=jax.ShapeDtypeStruct((_NTOK, 128), jnp.float32),
    scratch_types=[
        pltpu.VMEM((_CHUNK,), jnp.int32),
        pltpu.VMEM((_CHUNK,), jnp.int32),
        pltpu.VMEM((_CHUNK, 128), jnp.float32),
        pltpu.VMEM((_CHUNK, 128), jnp.float32),
        pltpu.SemaphoreType.DMA,
        pltpu.SemaphoreType.DMA,
        pltpu.SemaphoreType.DMA,
    ],
    compiler_params=pltpu.CompilerParams(use_tc_tiling_on_sc=False,
                                         needs_layout_passes=False),
)


_LN_SEQS = 128
_LN_BLK = _LN_SEQS * _MAXLEN


_LN_SUB = 16
_SUB_ROWS = _LN_SUB * _MAXLEN


def _ln_body(x_ref, pos_ref, gamma_ref, beta_ref, o_ref):
    avg = jnp.full((_HID, _HID), 1.0 / _HID, jnp.float32)
    for j in range(_LN_SEQS // _LN_SUB):
        x128 = x_ref[pl.ds(j * _SUB_ROWS, _SUB_ROWS), :]
        x = x128[:, :_HID] + pos_ref[...]
        mean = jnp.dot(x, avg, preferred_element_type=jnp.float32)
        ex2 = jnp.dot(x * x, avg, preferred_element_type=jnp.float32)
        var = ex2 - mean * mean
        normed = (x - mean) * lax.rsqrt(var + _EPS)
        y = normed * gamma_ref[...][None, :] + beta_ref[...][None, :]
        mask = x128[:, _HID:_HID + 1]
        y = y * mask
        o_ref[:, :, pl.ds(j * _LN_SUB, _LN_SUB)] = (
            y.reshape(_LN_SUB, _MAXLEN, _HID).transpose(1, 2, 0))


def _ln(rows, pos_rep, gamma, beta):
    grid = (_NTOK // _LN_BLK,)
    return pl.pallas_call(
        _ln_body,
        grid=grid,
        in_specs=[
            pl.BlockSpec((_LN_BLK, 128), lambda i: (i, 0)),
            pl.BlockSpec((_SUB_ROWS, _HID), lambda i: (0, 0)),
            pl.BlockSpec((_HID,), lambda i: (0,)),
            pl.BlockSpec((_HID,), lambda i: (0,)),
        ],
        out_specs=pl.BlockSpec((_MAXLEN, _HID, _LN_SEQS), lambda i: (0, 0, i)),
        out_shape=jax.ShapeDtypeStruct((_MAXLEN, _HID, _BATCH), jnp.float32),
    )(rows, pos_rep, gamma, beta)


@jax.jit
def _run(tok_flat, word_table, pos_rep, gamma, beta):
    wordp = jnp.pad(word_table, ((0, 0), (0, 128 - _HID)))
    rows = _gather(tok_flat, wordp)
    out = _ln(rows, pos_rep, gamma, beta)
    return jnp.transpose(out, (2, 0, 1))


def kernel(tokens, word_table, pos_table, gamma, beta):
    tok_flat = tokens.reshape(-1).astype(jnp.int32)
    pos_rep = jnp.tile(pos_table, (_LN_SUB, 1))
    return _run(tok_flat, word_table, pos_rep, gamma, beta)


# speedup vs baseline: 1.7300x; 1.7300x over previous
import jax
import jax.numpy as jnp
from jax import lax
from jax.experimental import pallas as pl
from jax.experimental.pallas import tpu as pltpu
from jax.experimental.pallas import tpu_sc as plsc

_VOCAB = 100000
_HID = 64
_MAXLEN = 200
_BATCH = 4096
_EPS = 1e-8
_NC = 2
_NS = 16
_NW = _NC * _NS
_NTOK = _BATCH * _MAXLEN
_TOK_PER_W = _NTOK // _NW
_CHUNK = 256
_NCHUNK = _TOK_PER_W // _CHUNK


def _gather_body(tok_hbm, word_hbm, rows_hbm,
                 idx0, idx1, buf0, buf1, sem0, sem1, semo):
    wid = lax.axis_index("s") * _NC + lax.axis_index("c")
    base = wid * _TOK_PER_W

    def issue(off, idx_v, buf_v, sem):
        pltpu.sync_copy(tok_hbm.at[pl.ds(off, _CHUNK)], idx_v)
        for j in range(_CHUNK // 128):
            pltpu.async_copy(word_hbm.at[idx_v.at[pl.ds(128 * j, 128)]],
                             buf_v.at[pl.ds(128 * j, 128)], sem)

    lane = lax.iota(jnp.int32, 16)
    col64 = jnp.full((16,), _HID, jnp.int32)

    def drain(off, idx_v, buf_v, sem):
        for j in range(_CHUNK // 128):
            pltpu.make_async_copy(word_hbm.at[idx_v.at[pl.ds(128 * j, 128)]],
                                  buf_v.at[pl.ds(128 * j, 128)], sem).wait()
        for k in range(_CHUNK // 16):
            tk = idx_v[pl.ds(16 * k, 16)]
            m = jnp.where(tk != 0, jnp.float32(1.0), jnp.float32(0.0))
            plsc.store_scatter(buf_v, [16 * k + lane, col64], m)
        pltpu.async_copy(buf_v, rows_hbm.at[pl.ds(off, _CHUNK)], semo).wait()

    issue(base, idx0, buf0, sem0)

    def chunk_body(i, c):
        slot = lax.rem(i, 2)

        @pl.when(jnp.logical_and(i + 1 < _NCHUNK, slot == 0))
        def _():
            issue(base + (i + 1) * _CHUNK, idx1, buf1, sem1)

        @pl.when(jnp.logical_and(i + 1 < _NCHUNK, slot == 1))
        def _():
            issue(base + (i + 1) * _CHUNK, idx0, buf0, sem0)

        off_i = base + i * _CHUNK

        @pl.when(slot == 0)
        def _():
            drain(off_i, idx0, buf0, sem0)

        @pl.when(slot == 1)
        def _():
            drain(off_i, idx1, buf1, sem1)

        return c

    lax.fori_loop(0, _NCHUNK, chunk_body, 0)


_gather = pl.kernel(
    _gather_body,
    mesh=plsc.VectorSubcoreMesh(core_axis_name="c", subcore_axis_name="s"),
    out_type=jax.ShapeDtypeStruct((_NTOK, 128), jnp.float32),
    scratch_types=[
        pltpu.VMEM((_CHUNK,), jnp.int32),
        pltpu.VMEM((_CHUNK,), jnp.int32),
        pltpu.VMEM((_CHUNK, 128), jnp.float32),
        pltpu.VMEM((_CHUNK, 128), jnp.float32),
        pltpu.SemaphoreType.DMA,
        pltpu.SemaphoreType.DMA,
        pltpu.SemaphoreType.DMA,
    ],
    compiler_params=pltpu.CompilerParams(use_tc_tiling_on_sc=False,
                                         needs_layout_passes=False),
)


_LN_SEQS = 64
_LN_BLK = _LN_SEQS * _MAXLEN


def _ln_body(x_ref, pos_ref, gamma_ref, beta_ref, o_ref):
    x128 = x_ref[...]
    x = x128[:, :_HID] + pos_ref[...]
    avg = jnp.full((_HID, _HID), 1.0 / _HID, jnp.float32)
    mean = jnp.dot(x, avg, preferred_element_type=jnp.float32)
    ex2 = jnp.dot(x * x, avg, preferred_element_type=jnp.float32)
    var = ex2 - mean * mean
    normed = (x - mean) * lax.rsqrt(var + _EPS)
    y = normed * gamma_ref[...][None, :] + beta_ref[...][None, :]
    mask = x128[:, _HID:_HID + 1]
    y = y * mask
    o_ref[...] = y.reshape(_LN_SEQS, _MAXLEN, _HID)


def _ln(rows, pos_rep, gamma, beta):
    grid = (_NTOK // _LN_BLK,)
    return pl.pallas_call(
        _ln_body,
        grid=grid,
        in_specs=[
            pl.BlockSpec((_LN_BLK, 128), lambda i: (i, 0)),
            pl.BlockSpec((_LN_BLK, _HID), lambda i: (0, 0)),
            pl.BlockSpec((_HID,), lambda i: (0,)),
            pl.BlockSpec((_HID,), lambda i: (0,)),
        ],
        out_specs=pl.BlockSpec((_LN_SEQS, _MAXLEN, _HID), lambda i: (i, 0, 0)),
        out_shape=jax.ShapeDtypeStruct((_BATCH, _MAXLEN, _HID), jnp.float32),
    )(rows, pos_rep, gamma, beta)


@jax.jit
def _run(tok_flat, word_table, pos_rep, gamma, beta):
    wordp = jnp.pad(word_table, ((0, 0), (0, 128 - _HID)))
    rows = _gather(tok_flat, wordp)
    return _ln(rows, pos_rep, gamma, beta)


def kernel(tokens, word_table, pos_table, gamma, beta):
    tok_flat = tokens.reshape(-1).astype(jnp.int32)
    pos_rep = jnp.tile(pos_table, (_LN_SEQS, 1))
    return _run(tok_flat, word_table, pos_rep, gamma, beta)


# speedup vs baseline: 1.7737x; 1.0252x over previous
import jax
import jax.numpy as jnp
from jax import lax
from jax.experimental import pallas as pl
from jax.experimental.pallas import tpu as pltpu
from jax.experimental.pallas import tpu_sc as plsc

_VOCAB = 100000
_HID = 64
_MAXLEN = 200
_BATCH = 4096
_EPS = 1e-8
_NC = 2
_NS = 16
_NW = _NC * _NS
_NTOK = _BATCH * _MAXLEN
_TOK_PER_W = _NTOK // _NW
_CHUNK = 256
_NCHUNK = _TOK_PER_W // _CHUNK


def _gather_body(tok_hbm, word_hbm, rows_hbm,
                 idx0, idx1, buf0, buf1, sem0, sem1, semo):
    wid = lax.axis_index("s") * _NC + lax.axis_index("c")
    base = wid * _TOK_PER_W

    def issue(off, idx_v, buf_v, sem):
        pltpu.sync_copy(tok_hbm.at[pl.ds(off, _CHUNK)], idx_v)
        for j in range(_CHUNK // 128):
            pltpu.async_copy(word_hbm.at[idx_v.at[pl.ds(128 * j, 128)]],
                             buf_v.at[pl.ds(128 * j, 128)], sem)

    lane = lax.iota(jnp.int32, 16)
    col64 = jnp.full((16,), _HID, jnp.int32)

    def drain(off, idx_v, buf_v, sem):
        for j in range(_CHUNK // 128):
            pltpu.make_async_copy(word_hbm.at[idx_v.at[pl.ds(128 * j, 128)]],
                                  buf_v.at[pl.ds(128 * j, 128)], sem).wait()
        for k in range(_CHUNK // 16):
            tk = idx_v[pl.ds(16 * k, 16)]
            m = jnp.where(tk != 0, jnp.float32(1.0), jnp.float32(0.0))
            plsc.store_scatter(buf_v, [16 * k + lane, col64], m)
        pltpu.async_copy(buf_v.at[:, pl.ds(0, 80)],
                         rows_hbm.at[pl.ds(off, _CHUNK), pl.ds(0, 80)],
                         semo).wait()

    issue(base, idx0, buf0, sem0)

    def chunk_body(i, c):
        slot = lax.rem(i, 2)

        @pl.when(jnp.logical_and(i + 1 < _NCHUNK, slot == 0))
        def _():
            issue(base + (i + 1) * _CHUNK, idx1, buf1, sem1)

        @pl.when(jnp.logical_and(i + 1 < _NCHUNK, slot == 1))
        def _():
            issue(base + (i + 1) * _CHUNK, idx0, buf0, sem0)

        off_i = base + i * _CHUNK

        @pl.when(slot == 0)
        def _():
            drain(off_i, idx0, buf0, sem0)

        @pl.when(slot == 1)
        def _():
            drain(off_i, idx1, buf1, sem1)

        return c

    lax.fori_loop(0, _NCHUNK, chunk_body, 0)


_gather = pl.kernel(
    _gather_body,
    mesh=plsc.VectorSubcoreMesh(core_axis_name="c", subcore_axis_name="s"),
    out_type=jax.ShapeDtypeStruct((_NTOK, 128), jnp.float32),
    scratch_types=[
        pltpu.VMEM((_CHUNK,), jnp.int32),
        pltpu.VMEM((_CHUNK,), jnp.int32),
        pltpu.VMEM((_CHUNK, 128), jnp.float32),
        pltpu.VMEM((_CHUNK, 128), jnp.float32),
        pltpu.SemaphoreType.DMA,
        pltpu.SemaphoreType.DMA,
        pltpu.SemaphoreType.DMA,
    ],
    compiler_params=pltpu.CompilerParams(use_tc_tiling_on_sc=False,
                                         needs_layout_passes=False),
)


_LN_SEQS = 64
_LN_BLK = _LN_SEQS * _MAXLEN


def _ln_body(x_ref, pos_ref, gamma_ref, beta_ref, o_ref):
    x128 = x_ref[...]
    x = x128[:, :_HID] + pos_ref[...]
    avg = jnp.full((_HID, _HID), 1.0 / _HID, jnp.float32)
    mean = jnp.dot(x, avg, preferred_element_type=jnp.float32)
    ex2 = jnp.dot(x * x, avg, preferred_element_type=jnp.float32)
    var = ex2 - mean * mean
    normed = (x - mean) * lax.rsqrt(var + _EPS)
    y = normed * gamma_ref[...][None, :] + beta_ref[...][None, :]
    mask = x128[:, _HID:_HID + 1]
    y = y * mask
    o_ref[...] = y.reshape(_LN_SEQS, _MAXLEN, _HID)


def _ln(rows, pos_rep, gamma, beta):
    grid = (_NTOK // _LN_BLK,)
    return pl.pallas_call(
        _ln_body,
        grid=grid,
        in_specs=[
            pl.BlockSpec((_LN_BLK, 128), lambda i: (i, 0)),
            pl.BlockSpec((_LN_BLK, _HID), lambda i: (0, 0)),
            pl.BlockSpec((_HID,), lambda i: (0,)),
            pl.BlockSpec((_HID,), lambda i: (0,)),
        ],
        out_specs=pl.BlockSpec((_LN_SEQS, _MAXLEN, _HID), lambda i: (i, 0, 0)),
        out_shape=jax.ShapeDtypeStruct((_BATCH, _MAXLEN, _HID), jnp.float32),
    )(rows, pos_rep, gamma, beta)


@jax.jit
def _run(tok_flat, word_table, pos_rep, gamma, beta):
    wordp = jnp.pad(word_table, ((0, 0), (0, 128 - _HID)))
    rows = _gather(tok_flat, wordp)
    return _ln(rows, pos_rep, gamma, beta)


def kernel(tokens, word_table, pos_table, gamma, beta):
    tok_flat = tokens.reshape(-1).astype(jnp.int32)
    pos_rep = jnp.tile(pos_table, (_LN_SEQS, 1))
    return _run(tok_flat, word_table, pos_rep, gamma, beta)
